# Initial kernel scaffold; baseline (speedup 1.0000x reference)
import functools

import jax
import jax.numpy as jnp
from jax import lax
from jax.experimental import pallas as pl
from jax.experimental.pallas import tpu as pltpu
from jax.experimental.pallas import tpu_sc as plsc

B = 16384
M = 50
E = 32
N_KEYS = 1_000_000
NULL_KEY = 0

NC, NS, L = 2, 16, 16
NW = NC * NS
ROWS_W = B // NW
CB = 32
NCHUNK = ROWS_W // CB
G = 16
GS = CB * M // G
RB = B // 2
CR = CB // 2
INV_M = 1.0 / M


def _zero16():
    return jnp.zeros((L,), jnp.float32)


@functools.cache
def _build():
    mesh = plsc.VectorSubcoreMesh(core_axis_name="c", subcore_axis_name="s")

    @functools.partial(
        pl.kernel,
        out_type=jax.ShapeDtypeStruct((B, E), jnp.float32),
        mesh=mesh,
        scratch_types=[
            pltpu.VMEM((G, GS), jnp.int32),
            pltpu.VMEM((G, GS), jnp.int32),
            pltpu.VMEM((G, GS, E), jnp.float32),
            pltpu.VMEM((G, GS, E), jnp.float32),
            pltpu.VMEM((M, E), jnp.float32),
            pltpu.VMEM((CB, E), jnp.float32),
            pltpu.VMEM((CB, E), jnp.float32),
            pltpu.SemaphoreType.DMA,
            pltpu.SemaphoreType.DMA,
            pltpu.SemaphoreType.DMA,
            pltpu.SemaphoreType.DMA,
            pltpu.SemaphoreType.DMA,
            pltpu.SemaphoreType.DMA,
        ],
    )
    def body(inp_hbm, table_hbm, de_hbm, out_hbm,
             idx0, idx1, rows0, rows1, de_v, acc0, acc1,
             sem_g0, sem_g1, sem_i0, sem_i1, sem_o0, sem_o1):
        wid = lax.axis_index("s") * NC + lax.axis_index("c")
        ibase = wid * (RB // NW)
        obase = wid * ROWS_W

        idxs = (idx0, idx1)
        rows = (rows0, rows1)
        accs = (acc0, acc1)
        sem_g = (sem_g0, sem_g1)
        sem_i = (sem_i0, sem_i1)
        sem_o = (sem_o0, sem_o1)

        def idx_src(n):
            return inp_hbm.at[pl.ds(ibase + n * CR, CR)]

        def out_dst(n):
            return out_hbm.at[pl.ds(obase + n * CB, CB)]

        def fire_gathers(p):
            for g in range(G):
                pltpu.async_copy(table_hbm.at[idxs[p].at[g]],
                                 rows[p].at[g], sem_g[p])

        def drain_gathers(p):
            for g in range(G):
                pltpu.make_async_copy(table_hbm.at[idxs[p].at[g]],
                                      rows[p].at[g], sem_g[p]).wait()

        pltpu.sync_copy(de_hbm, de_v)

        def scale_de(m, c):
            de_v[m, pl.ds(0, L)] = de_v[m, pl.ds(0, L)] * INV_M
            de_v[m, pl.ds(L, L)] = de_v[m, pl.ds(L, L)] * INV_M
            return c
        lax.fori_loop(0, M, scale_de, 0)

        pltpu.async_copy(idx_src(0), idxs[0], sem_i[0])
        pltpu.async_copy(idx_src(1), idxs[1], sem_i[1])
        pltpu.make_async_copy(idx_src(0), idxs[0], sem_i[0]).wait()
        fire_gathers(0)

        def null_fix(p):
            mn = idxs[p][0, pl.ds(0, L)]
            for g in range(G):
                for s in (0, 16, 32, 48, 64, 80, GS - L):
                    mn = jnp.minimum(mn, idxs[p][g, pl.ds(s, L)])

            @pl.when(jnp.min(mn) == NULL_KEY)
            def _():
                def fix_g(g, c):
                    def fix_r(r, c2):
                        @pl.when(idxs[p][g, r] == NULL_KEY)
                        def _():
                            rows[p][g, r, pl.ds(0, L)] = _zero16()
                            rows[p][g, r, pl.ds(L, L)] = _zero16()
                        return c2
                    lax.fori_loop(0, GS, fix_r, 0)
                    return c
                lax.fori_loop(0, G, fix_g, 0)

        def compute(p):
            acc = accs[p]

            def zero_b(b, c):
                acc[b, pl.ds(0, L)] = _zero16()
                acc[b, pl.ds(L, L)] = _zero16()
                return c
            lax.fori_loop(0, CB, zero_b, 0)

            def mbody(m, c):
                d0 = de_v[m, pl.ds(0, L)]
                d1 = de_v[m, pl.ds(L, L)]
                for a in range(G):
                    plsc.addupdate(acc.at[2 * a, pl.ds(0, L)],
                                   rows[p][a, m, pl.ds(0, L)] * d0)
                    plsc.addupdate(acc.at[2 * a, pl.ds(L, L)],
                                   rows[p][a, m, pl.ds(L, L)] * d1)
                    plsc.addupdate(acc.at[2 * a + 1, pl.ds(0, L)],
                                   rows[p][a, M + m, pl.ds(0, L)] * d0)
                    plsc.addupdate(acc.at[2 * a + 1, pl.ds(L, L)],
                                   rows[p][a, M + m, pl.ds(L, L)] * d1)
                return c
            lax.fori_loop(0, M, mbody, 0)

        def chunk_step(n, p):
            drain_gathers(p)
            q = 1 - p

            @pl.when(n + 1 < NCHUNK)
            def _():
                pltpu.make_async_copy(idx_src(n + 1), idxs[q], sem_i[q]).wait()
                fire_gathers(q)

            null_fix(p)

            @pl.when(n + 2 < NCHUNK)
            def _():
                pltpu.async_copy(idx_src(n + 2), idxs[p], sem_i[p])

            @pl.when(n >= 2)
            def _():
                pltpu.make_async_copy(accs[p], out_dst(n - 2), sem_o[p]).wait()

            compute(p)
            pltpu.async_copy(accs[p], out_dst(n), sem_o[p])

        def pair(k, c):
            n = k * 2
            chunk_step(n, 0)
            chunk_step(n + 1, 1)
            return c
        lax.fori_loop(0, NCHUNK // 2, pair, 0)

        pltpu.make_async_copy(accs[0], out_dst(NCHUNK - 2), sem_o[0]).wait()
        pltpu.make_async_copy(accs[1], out_dst(NCHUNK - 1), sem_o[1]).wait()

    return body


def kernel(input, table, depth_embedding):
    inp = input.astype(jnp.int32).reshape(RB, B * M // RB)
    return _build()(inp, table, depth_embedding)


# speedup vs baseline: 2.2772x; 2.2772x over previous
import functools

import jax
import jax.numpy as jnp
from jax import lax
from jax.experimental import pallas as pl
from jax.experimental.pallas import tpu as pltpu
from jax.experimental.pallas import tpu_sc as plsc

B = 16384
M = 50
E = 32
N_KEYS = 1_000_000
NULL_KEY = 0

NC, NS, L = 2, 16, 16
NW = NC * NS
ROWS_W = B // NW
CB = 32
NCHUNK = ROWS_W // CB
G = 16
GS = CB * M // G
RB = B // 2
CR = CB // 2
INV_M = 1.0 / M


def _zero16():
    return jnp.zeros((L,), jnp.float32)


@functools.cache
def _build():
    mesh = plsc.VectorSubcoreMesh(core_axis_name="c", subcore_axis_name="s")

    @functools.partial(
        pl.kernel,
        out_type=jax.ShapeDtypeStruct((B, E), jnp.float32),
        mesh=mesh,
        compiler_params=pltpu.CompilerParams(needs_layout_passes=False,
                                             use_tc_tiling_on_sc=False),
        scratch_types=[
            pltpu.VMEM((G, GS), jnp.int32),
            pltpu.VMEM((G, GS), jnp.int32),
            pltpu.VMEM((G, GS, E), jnp.float32),
            pltpu.VMEM((G, GS, E), jnp.float32),
            pltpu.VMEM((M, E), jnp.float32),
            pltpu.VMEM((CB, E), jnp.float32),
            pltpu.VMEM((CB, E), jnp.float32),
            pltpu.SemaphoreType.DMA,
            pltpu.SemaphoreType.DMA,
            pltpu.SemaphoreType.DMA,
            pltpu.SemaphoreType.DMA,
            pltpu.SemaphoreType.DMA,
            pltpu.SemaphoreType.DMA,
        ],
    )
    def body(inp_hbm, table_hbm, de_hbm, out_hbm,
             idx0, idx1, rows0, rows1, de_v, acc0, acc1,
             sem_g0, sem_g1, sem_i0, sem_i1, sem_o0, sem_o1):
        wid = lax.axis_index("s") * NC + lax.axis_index("c")
        ibase = wid * (RB // NW)
        obase = wid * ROWS_W

        idxs = (idx0, idx1)
        rows = (rows0, rows1)
        accs = (acc0, acc1)
        sem_g = (sem_g0, sem_g1)
        sem_i = (sem_i0, sem_i1)
        sem_o = (sem_o0, sem_o1)

        def idx_src(n):
            return inp_hbm.at[pl.ds(ibase + n * CR, CR)]

        def out_dst(n):
            return out_hbm.at[pl.ds(obase + n * CB, CB)]

        def fire_gathers(p):
            for g in range(G):
                pltpu.async_copy(table_hbm.at[idxs[p].at[g]],
                                 rows[p].at[g], sem_g[p])

        def drain_gathers(p):
            for g in range(G):
                pltpu.make_async_copy(table_hbm.at[idxs[p].at[g]],
                                      rows[p].at[g], sem_g[p]).wait()

        pltpu.sync_copy(de_hbm, de_v)

        def scale_de(m, c):
            de_v[m, pl.ds(0, L)] = de_v[m, pl.ds(0, L)] * INV_M
            de_v[m, pl.ds(L, L)] = de_v[m, pl.ds(L, L)] * INV_M
            return c
        lax.fori_loop(0, M, scale_de, 0)

        pltpu.async_copy(idx_src(0), idxs[0], sem_i[0])
        pltpu.async_copy(idx_src(1), idxs[1], sem_i[1])
        pltpu.make_async_copy(idx_src(0), idxs[0], sem_i[0]).wait()
        fire_gathers(0)

        def null_fix(p):
            mn = idxs[p][0, pl.ds(0, L)]
            for g in range(G):
                for s in (0, 16, 32, 48, 64, 80, GS - L):
                    mn = jnp.minimum(mn, idxs[p][g, pl.ds(s, L)])

            nnull = plsc.all_reduce_population_count(mn == NULL_KEY)

            @pl.when(nnull[0] > 0)
            def _():
                def fix_g(g, c):
                    def fix_s(si, c2):
                        start = jnp.minimum(si * L, GS - L)
                        v16 = idxs[p][g, pl.ds(start, L)]
                        for l in range(L):
                            @pl.when(v16[l] == NULL_KEY)
                            def _():
                                r = start + l
                                rows[p][g, r, pl.ds(0, L)] = _zero16()
                                rows[p][g, r, pl.ds(L, L)] = _zero16()
                        return c2
                    lax.fori_loop(0, (GS + L - 1) // L, fix_s, 0)
                    return c
                lax.fori_loop(0, G, fix_g, 0)

        def compute(p):
            acc = accs[p]

            def zero_b(b, c):
                acc[b, pl.ds(0, L)] = _zero16()
                acc[b, pl.ds(L, L)] = _zero16()
                return c
            lax.fori_loop(0, CB, zero_b, 0)

            def mbody(m, c):
                d0 = de_v[m, pl.ds(0, L)]
                d1 = de_v[m, pl.ds(L, L)]
                for a in range(G):
                    plsc.addupdate(acc.at[2 * a, pl.ds(0, L)],
                                   rows[p][a, m, pl.ds(0, L)] * d0)
                    plsc.addupdate(acc.at[2 * a, pl.ds(L, L)],
                                   rows[p][a, m, pl.ds(L, L)] * d1)
                    plsc.addupdate(acc.at[2 * a + 1, pl.ds(0, L)],
                                   rows[p][a, M + m, pl.ds(0, L)] * d0)
                    plsc.addupdate(acc.at[2 * a + 1, pl.ds(L, L)],
                                   rows[p][a, M + m, pl.ds(L, L)] * d1)
                return c
            lax.fori_loop(0, M, mbody, 0)

        def chunk_step(n, p):
            drain_gathers(p)
            q = 1 - p

            @pl.when(n + 1 < NCHUNK)
            def _():
                pltpu.make_async_copy(idx_src(n + 1), idxs[q], sem_i[q]).wait()
                fire_gathers(q)

            null_fix(p)

            @pl.when(n + 2 < NCHUNK)
            def _():
                pltpu.async_copy(idx_src(n + 2), idxs[p], sem_i[p])

            @pl.when(n >= 2)
            def _():
                pltpu.make_async_copy(accs[p], out_dst(n - 2), sem_o[p]).wait()

            compute(p)
            pltpu.async_copy(accs[p], out_dst(n), sem_o[p])

        def pair(k, c):
            n = k * 2
            chunk_step(n, 0)
            chunk_step(n + 1, 1)
            return c
        lax.fori_loop(0, NCHUNK // 2, pair, 0)

        pltpu.make_async_copy(accs[0], out_dst(NCHUNK - 2), sem_o[0]).wait()
        pltpu.make_async_copy(accs[1], out_dst(NCHUNK - 1), sem_o[1]).wait()

    return body


def kernel(input, table, depth_embedding):
    inp = input.astype(jnp.int32).reshape(RB, B * M // RB)
    return _build()(inp, table, depth_embedding)


# speedup vs baseline: 2.9173x; 1.2811x over previous
import functools

import jax
import jax.numpy as jnp
from jax import lax
from jax.experimental import pallas as pl
from jax.experimental.pallas import tpu as pltpu
from jax.experimental.pallas import tpu_sc as plsc

B = 16384
M = 50
E = 32
N_KEYS = 1_000_000
NULL_KEY = 0

NC, NS, L = 2, 16, 16
NW = NC * NS
ROWS_W = B // NW
CB = 32
NCHUNK = ROWS_W // CB
INV_M = 1.0 / M

M_SLICES = (0, 16, 32, M - L)


def _zero16():
    return jnp.zeros((L,), jnp.float32)


@functools.cache
def _build():
    mesh = plsc.VectorSubcoreMesh(core_axis_name="c", subcore_axis_name="s")

    @functools.partial(
        pl.kernel,
        out_type=jax.ShapeDtypeStruct((B, E), jnp.float32),
        mesh=mesh,
        compiler_params=pltpu.CompilerParams(needs_layout_passes=False,
                                             use_tc_tiling_on_sc=False),
        scratch_types=[
            pltpu.VMEM((CB, M), jnp.int32),
            pltpu.VMEM((CB, M), jnp.int32),
            pltpu.VMEM((CB, M, E), jnp.float32),
            pltpu.VMEM((CB, M, E), jnp.float32),
            pltpu.VMEM((M, E), jnp.float32),
            pltpu.VMEM((CB, E), jnp.float32),
            pltpu.VMEM((CB, E), jnp.float32),
            pltpu.SemaphoreType.DMA,
            pltpu.SemaphoreType.DMA,
            pltpu.SemaphoreType.DMA,
            pltpu.SemaphoreType.DMA,
            pltpu.SemaphoreType.DMA,
            pltpu.SemaphoreType.DMA,
        ],
    )
    def body(inp_hbm, table_hbm, de_hbm, out_hbm,
             idx0, idx1, rows0, rows1, de_v, acc0, acc1,
             sem_g0, sem_g1, sem_i0, sem_i1, sem_o0, sem_o1):
        wid = lax.axis_index("s") * NC + lax.axis_index("c")
        base = wid * ROWS_W

        idxs = (idx0, idx1)
        rows = (rows0, rows1)
        accs = (acc0, acc1)
        sem_g = (sem_g0, sem_g1)
        sem_i = (sem_i0, sem_i1)
        sem_o = (sem_o0, sem_o1)

        def blk(n):
            return pl.ds(base + n * CB, CB)

        def fire_gather(p):
            for b in range(CB):
                pltpu.async_copy(table_hbm.at[idxs[p].at[b]],
                                 rows[p].at[b], sem_g[p])

        def drain_gather(p):
            for b in range(CB):
                pltpu.make_async_copy(table_hbm.at[idxs[p].at[b]],
                                      rows[p].at[b], sem_g[p]).wait()

        pltpu.sync_copy(de_hbm, de_v)

        def scale_de(m, c):
            de_v[m, pl.ds(0, L)] = de_v[m, pl.ds(0, L)] * INV_M
            de_v[m, pl.ds(L, L)] = de_v[m, pl.ds(L, L)] * INV_M
            return c
        lax.fori_loop(0, M, scale_de, 0)

        pltpu.async_copy(inp_hbm.at[blk(0)], idxs[0], sem_i[0])
        pltpu.async_copy(inp_hbm.at[blk(1)], idxs[1], sem_i[1])
        pltpu.make_async_copy(inp_hbm.at[blk(0)], idxs[0], sem_i[0]).wait()
        fire_gather(0)

        def null_fix(p):
            mn = idxs[p][0, pl.ds(0, L)]
            for b in range(CB):
                for s in M_SLICES:
                    mn = jnp.minimum(mn, idxs[p][b, pl.ds(s, L)])
            nnull = plsc.all_reduce_population_count(mn == NULL_KEY)

            @pl.when(nnull[0] > 0)
            def _():
                def fix_b(b, c):
                    def fix_s(si, c2):
                        start = jnp.minimum(si * L, M - L)
                        v16 = idxs[p][b, pl.ds(start, L)]
                        for l in range(L):
                            @pl.when(v16[l] == NULL_KEY)
                            def _():
                                r = start + l
                                rows[p][b, r, pl.ds(0, L)] = _zero16()
                                rows[p][b, r, pl.ds(L, L)] = _zero16()
                        return c2
                    lax.fori_loop(0, len(M_SLICES), fix_s, 0)
                    return c
                lax.fori_loop(0, CB, fix_b, 0)

        def compute(p):
            acc = accs[p]

            def zero_b(b, c):
                acc[b, pl.ds(0, L)] = _zero16()
                acc[b, pl.ds(L, L)] = _zero16()
                return c
            lax.fori_loop(0, CB, zero_b, 0)

            def mbody(m, c):
                d0 = de_v[m, pl.ds(0, L)]
                d1 = de_v[m, pl.ds(L, L)]
                for b0 in range(0, CB, 8):
                    items = []
                    for b in range(b0, b0 + 8):
                        items.append((b, 0, rows[p][b, m, pl.ds(0, L)], d0))
                        items.append((b, L, rows[p][b, m, pl.ds(L, L)], d1))
                    prods = [(b, off, r * d) for (b, off, r, d) in items]
                    for b, off, pv in prods:
                        plsc.addupdate(acc.at[b, pl.ds(off, L)], pv)
                return c
            lax.fori_loop(0, M, mbody, 0)

        def chunk_step(n, p):
            drain_gather(p)
            q = 1 - p

            @pl.when(n + 1 < NCHUNK)
            def _():
                pltpu.make_async_copy(inp_hbm.at[blk(n + 1)], idxs[q],
                                      sem_i[q]).wait()
                fire_gather(q)

            null_fix(p)

            @pl.when(n + 2 < NCHUNK)
            def _():
                pltpu.async_copy(inp_hbm.at[blk(n + 2)], idxs[p], sem_i[p])

            @pl.when(n >= 2)
            def _():
                pltpu.make_async_copy(accs[p], out_hbm.at[blk(n - 2)],
                                      sem_o[p]).wait()

            compute(p)
            pltpu.async_copy(accs[p], out_hbm.at[blk(n)], sem_o[p])

        def pair(k, c):
            n = k * 2
            chunk_step(n, 0)
            chunk_step(n + 1, 1)
            return c
        lax.fori_loop(0, NCHUNK // 2, pair, 0)

        pltpu.make_async_copy(accs[0], out_hbm.at[blk(NCHUNK - 2)],
                              sem_o[0]).wait()
        pltpu.make_async_copy(accs[1], out_hbm.at[blk(NCHUNK - 1)],
                              sem_o[1]).wait()

    return body


def kernel(input, table, depth_embedding):
    return _build()(input.astype(jnp.int32), table, depth_embedding)
